# baseline (speedup 1.0000x reference)
import functools

import jax
import jax.numpy as jnp
from jax import lax
from jax.experimental import pallas as pl
from jax.experimental.pallas import tpu as pltpu
from jax.experimental.pallas import tpu_sc as plsc

R = 32
SEQ, DM = 2048, 1024
TS, TD = 8, 128
NTD = DM // TD
NC, NS = 2, 16
NW = NC * NS
L = 16
VPT = TS * TD // L

S_SC = 704
NTILES = (S_SC // TS) * NTD
NBLK = NTILES // NW
TBS, DMB = 64, 1024

_VALLEY5 = ((0, 4), (1, 3), (1, 4), (2, 4), (3, 4))


def _ce(a, b):
    return jnp.maximum(a, b), jnp.minimum(a, b)


def _sort4(e):
    a0, a1 = _ce(e[0], e[1])
    a2, a3 = _ce(e[2], e[3])
    a0, a2 = _ce(a0, a2)
    a1, a3 = _ce(a1, a3)
    a1, a2 = _ce(a1, a2)
    return a0, a1, a2, a3


def _top5_of8(e):
    a = _sort4(e[0:4])
    b = _sort4(e[4:8])
    top, lo = [], []
    for i in range(4):
        hi, l = _ce(a[i], b[3 - i])
        top.append(hi)
        lo.append(l)
    u0, u2 = _ce(top[0], top[2])
    u1, u3 = _ce(top[1], top[3])
    t0, t1 = _ce(u0, u1)
    t2, t3 = _ce(u2, u3)
    g5 = jnp.maximum(jnp.maximum(lo[0], lo[1]), jnp.maximum(lo[2], lo[3]))
    return [t0, t1, t2, t3, g5]


def _merge5(p, q):
    t = [jnp.maximum(p[i], q[4 - i]) for i in range(5)]
    for i, j in _VALLEY5:
        t[i], t[j] = _ce(t[i], t[j])
    return t


def _top5_quantile(rows):
    g = [_top5_of8(rows[i * 8:(i + 1) * 8]) for i in range(4)]
    p = _merge5(g[0], g[1])
    q = _merge5(g[2], g[3])
    v28 = jnp.maximum(
        jnp.maximum(jnp.minimum(p[0], q[2]), jnp.minimum(p[1], q[1])),
        jnp.maximum(jnp.minimum(p[2], q[0]), jnp.maximum(p[3], q[3])))
    v27 = jnp.maximum(
        jnp.maximum(jnp.maximum(jnp.minimum(p[0], q[3]),
                                jnp.minimum(p[1], q[2])),
                    jnp.maximum(jnp.minimum(p[2], q[1]),
                                jnp.minimum(p[3], q[0]))),
        jnp.maximum(p[4], q[4]))
    return v27 + jnp.float32(0.9) * (v28 - v27)


def _qkernel(x_hbm, out_hbm, buf, obuf, insem, outsem):
    wid = lax.axis_index("s") * NC + lax.axis_index("c")
    tbase = wid * NBLK

    def in_copy(g, slot):
        t = tbase + g
        s0 = (t // NTD) * TS
        d0 = (t % NTD) * TD
        return pltpu.make_async_copy(
            x_hbm.at[:, pl.ds(s0, TS), pl.ds(d0, TD)], buf.at[slot],
            insem.at[slot])

    def out_copy(g, slot):
        t = tbase + g
        s0 = (t // NTD) * TS
        d0 = (t % NTD) * TD
        return pltpu.make_async_copy(
            obuf.at[slot], out_hbm.at[pl.ds(s0, TS), pl.ds(d0, TD)],
            outsem.at[slot])

    def compute(slot):
        @plsc.parallel_loop(0, VPT, unroll=8)
        def jbody(j):
            i = j // (TD // L)
            c = (j % (TD // L)) * L
            rows = [buf[slot, r, i, pl.ds(c, L)] for r in range(R)]
            obuf[slot, i, pl.ds(c, L)] = _top5_quantile(rows)

    in_copy(0, 0).start()
    in_copy(1, 1).start()

    def gbody(i, carry):
        g = i * 2
        for slot in range(2):
            gs = g + slot
            in_copy(gs, slot).wait()

            @pl.when(gs >= 2)
            def _():
                out_copy(gs - 2, slot).wait()

            compute(slot)
            out_copy(gs, slot).start()

            @pl.when(gs + 2 < NBLK)
            def _():
                in_copy(gs + 2, slot).start()

        return carry

    lax.fori_loop(0, NBLK // 2, gbody, 0)
    out_copy(NBLK - 2, 0).wait()
    out_copy(NBLK - 1, 1).wait()


def _tc_body(x_ref, o_ref):
    rows = [x_ref[r] for r in range(R)]
    o_ref[...] = _top5_quantile(rows)


def kernel(tensor_list):
    mesh = plsc.VectorSubcoreMesh(core_axis_name="c", subcore_axis_name="s")
    run = functools.partial(
        pl.kernel,
        out_type=jax.ShapeDtypeStruct((S_SC, DM), jnp.float32),
        mesh=mesh,
        compiler_params=pltpu.CompilerParams(use_tc_tiling_on_sc=True),
        scratch_types=[
            pltpu.VMEM((2, R, TS, TD), jnp.float32),
            pltpu.VMEM((2, TS, TD), jnp.float32),
            pltpu.SemaphoreType.DMA((2,)),
            pltpu.SemaphoreType.DMA((2,)),
        ],
    )(_qkernel)
    out_sc = run(tensor_list)

    n_tc = SEQ - S_SC
    out_tc = pl.pallas_call(
        _tc_body,
        grid=(n_tc // TBS, DM // DMB),
        in_specs=[pl.BlockSpec(
            (R, TBS, DMB), lambda i, j: (0, S_SC // TBS + i, j))],
        out_specs=pl.BlockSpec(
            (TBS, DMB), lambda i, j: (S_SC // TBS + i, j)),
        out_shape=jax.ShapeDtypeStruct((SEQ, DM), jnp.float32),
    )(tensor_list)
    return lax.dynamic_update_slice(out_tc, out_sc, (0, 0))


# speedup vs baseline: 1.0129x; 1.0129x over previous
import functools

import jax
import jax.numpy as jnp
from jax import lax
from jax.experimental import pallas as pl
from jax.experimental.pallas import tpu as pltpu
from jax.experimental.pallas import tpu_sc as plsc

R = 32
SEQ, DM = 2048, 1024
TS, TD = 8, 128
NTD = DM // TD
NC, NS = 2, 16
NW = NC * NS
L = 16
VPT = TS * TD // L

S_SC = 768
NTILES = (S_SC // TS) * NTD
NBLK = NTILES // NW
TBS, DMB = 64, 1024

_VALLEY5 = ((0, 4), (1, 3), (1, 4), (2, 4), (3, 4))


def _ce(a, b):
    return jnp.maximum(a, b), jnp.minimum(a, b)


def _sort4(e):
    a0, a1 = _ce(e[0], e[1])
    a2, a3 = _ce(e[2], e[3])
    a0, a2 = _ce(a0, a2)
    a1, a3 = _ce(a1, a3)
    a1, a2 = _ce(a1, a2)
    return a0, a1, a2, a3


def _top5_of8(e):
    a = _sort4(e[0:4])
    b = _sort4(e[4:8])
    top, lo = [], []
    for i in range(4):
        hi, l = _ce(a[i], b[3 - i])
        top.append(hi)
        lo.append(l)
    u0, u2 = _ce(top[0], top[2])
    u1, u3 = _ce(top[1], top[3])
    t0, t1 = _ce(u0, u1)
    t2, t3 = _ce(u2, u3)
    g5 = jnp.maximum(jnp.maximum(lo[0], lo[1]), jnp.maximum(lo[2], lo[3]))
    return [t0, t1, t2, t3, g5]


def _merge5(p, q):
    t = [jnp.maximum(p[i], q[4 - i]) for i in range(5)]
    for i, j in _VALLEY5:
        t[i], t[j] = _ce(t[i], t[j])
    return t


def _top5_quantile(rows):
    g = [_top5_of8(rows[i * 8:(i + 1) * 8]) for i in range(4)]
    p = _merge5(g[0], g[1])
    q = _merge5(g[2], g[3])
    v28 = jnp.maximum(
        jnp.maximum(jnp.minimum(p[0], q[2]), jnp.minimum(p[1], q[1])),
        jnp.maximum(jnp.minimum(p[2], q[0]), jnp.maximum(p[3], q[3])))
    v27 = jnp.maximum(
        jnp.maximum(jnp.maximum(jnp.minimum(p[0], q[3]),
                                jnp.minimum(p[1], q[2])),
                    jnp.maximum(jnp.minimum(p[2], q[1]),
                                jnp.minimum(p[3], q[0]))),
        jnp.maximum(p[4], q[4]))
    return v27 + jnp.float32(0.9) * (v28 - v27)


def _qkernel(x_hbm, out_hbm, buf, obuf, insem, outsem):
    wid = lax.axis_index("s") * NC + lax.axis_index("c")
    tbase = wid * NBLK

    def in_copy(g, slot):
        t = tbase + g
        s0 = (t // NTD) * TS
        d0 = (t % NTD) * TD
        return pltpu.make_async_copy(
            x_hbm.at[:, pl.ds(s0, TS), pl.ds(d0, TD)], buf.at[slot],
            insem.at[slot])

    def out_copy(g, slot):
        t = tbase + g
        s0 = (t // NTD) * TS
        d0 = (t % NTD) * TD
        return pltpu.make_async_copy(
            obuf.at[slot], out_hbm.at[pl.ds(s0, TS), pl.ds(d0, TD)],
            outsem.at[slot])

    def compute(slot):
        @plsc.parallel_loop(0, VPT, unroll=8)
        def jbody(j):
            i = j // (TD // L)
            c = (j % (TD // L)) * L
            rows = [buf[slot, r, i, pl.ds(c, L)] for r in range(R)]
            obuf[slot, i, pl.ds(c, L)] = _top5_quantile(rows)

    in_copy(0, 0).start()
    in_copy(1, 1).start()

    def gbody(i, carry):
        g = i * 2
        for slot in range(2):
            gs = g + slot
            in_copy(gs, slot).wait()

            @pl.when(gs >= 2)
            def _():
                out_copy(gs - 2, slot).wait()

            compute(slot)
            out_copy(gs, slot).start()

            @pl.when(gs + 2 < NBLK)
            def _():
                in_copy(gs + 2, slot).start()

        return carry

    lax.fori_loop(0, NBLK // 2, gbody, 0)
    out_copy(NBLK - 2, 0).wait()
    out_copy(NBLK - 1, 1).wait()


def _tc_body(x_ref, o_ref):
    rows = [x_ref[r] for r in range(R)]
    o_ref[...] = _top5_quantile(rows)


def kernel(tensor_list):
    mesh = plsc.VectorSubcoreMesh(core_axis_name="c", subcore_axis_name="s")
    run = functools.partial(
        pl.kernel,
        out_type=jax.ShapeDtypeStruct((S_SC, DM), jnp.float32),
        mesh=mesh,
        compiler_params=pltpu.CompilerParams(use_tc_tiling_on_sc=True),
        scratch_types=[
            pltpu.VMEM((2, R, TS, TD), jnp.float32),
            pltpu.VMEM((2, TS, TD), jnp.float32),
            pltpu.SemaphoreType.DMA((2,)),
            pltpu.SemaphoreType.DMA((2,)),
        ],
    )(_qkernel)
    out_sc = run(tensor_list)

    n_tc = SEQ - S_SC
    out_tc = pl.pallas_call(
        _tc_body,
        grid=(n_tc // TBS, DM // DMB),
        in_specs=[pl.BlockSpec(
            (R, TBS, DMB), lambda i, j: (0, S_SC // TBS + i, j))],
        out_specs=pl.BlockSpec(
            (TBS, DMB), lambda i, j: (S_SC // TBS + i, j)),
        out_shape=jax.ShapeDtypeStruct((SEQ, DM), jnp.float32),
    )(tensor_list)
    return lax.dynamic_update_slice(out_tc, out_sc, (0, 0))
